# baseline (speedup 1.0000x reference)
import functools

import jax
import jax.numpy as jnp
from jax import lax
from jax.experimental import pallas as pl
from jax.experimental.pallas import tpu as pltpu
from jax.experimental.pallas import tpu_sc as plsc

SEQ = 200
B = 1024
V = 100000
D = 64
H = 64

NC = 2
NS = 16
NW = NC * NS
CHUNK = 128
PER_STEP = B // CHUNK
BASE_STEPS = SEQ // NW
EXTRA = SEQ - BASE_STEPS * NW


def _sc_gather(table, x):
    mesh = plsc.VectorSubcoreMesh(core_axis_name="c", subcore_axis_name="s")

    @functools.partial(
        pl.kernel,
        out_type=jax.ShapeDtypeStruct((SEQ, B, D), jnp.float32),
        mesh=mesh,
        scratch_types=[
            pltpu.VMEM((B,), jnp.int32),
            pltpu.VMEM((B, D), jnp.float32),
            pltpu.SemaphoreType.DMA,
        ],
        compiler_params=pltpu.CompilerParams(use_tc_tiling_on_sc=False),
    )
    def k(table_hbm, idx_hbm, out_hbm, idx_v, rows_v, sem):
        wid = lax.axis_index("s") * NC + lax.axis_index("c")
        t0 = jnp.where(wid < NW - EXTRA,
                       BASE_STEPS * wid,
                       BASE_STEPS * wid + (wid - (NW - EXTRA)))
        t1 = t0 + jnp.where(wid < NW - EXTRA, BASE_STEPS, BASE_STEPS + 1)

        @pl.loop(t0, t1)
        def step(t):
            pltpu.sync_copy(idx_hbm.at[t], idx_v)
            copies = [
                pltpu.async_copy(
                    table_hbm.at[idx_v.at[pl.ds(j * CHUNK, CHUNK)]],
                    rows_v.at[pl.ds(j * CHUNK, CHUNK)],
                    sem,
                )
                for j in range(PER_STEP)
            ]
            for c in copies:
                c.wait()
            pltpu.sync_copy(rows_v, out_hbm.at[t])

    return k(table, x)


T_BLK = 8
N_TBLK = SEQ // T_BLK


def _tc_gru(emb, w_ih, w_hh, brz, bin_, bhn, interpret=False):
    rhs_t = (((1,), (1,)), ((), ()))

    def body(emb_ref, wih_ref, whh_ref, brz_ref, bin_ref, bhn_ref,
             out_ref, h_ref):
        t = pl.program_id(0)

        @pl.when(t == 0)
        def _():
            h_ref[...] = jnp.zeros_like(h_ref)

        wih = wih_ref[...]
        whh = whh_ref[...]
        for i in range(T_BLK):
            h = h_ref[...]
            gi = jax.lax.dot_general(
                wih, emb_ref[i], rhs_t, preferred_element_type=jnp.float32)
            gh = jnp.dot(whh, h, preferred_element_type=jnp.float32)
            s = gi[: 2 * H] + gh[: 2 * H] + brz_ref[...]
            rz = 0.5 * jnp.tanh(0.5 * s) + 0.5
            r = rz[:H]
            z = rz[H:]
            n = jnp.tanh(gi[2 * H :] + bin_ref[...]
                         + r * (gh[2 * H :] + bhn_ref[...]))
            h_new = n + z * (h - n)
            h_ref[...] = h_new

        @pl.when(t == N_TBLK - 1)
        def _():
            out_ref[...] = h_ref[...]

    return pl.pallas_call(
        body,
        grid=(N_TBLK,),
        in_specs=[
            pl.BlockSpec((T_BLK, B, D), lambda t: (t, 0, 0)),
            pl.BlockSpec((3 * H, D), lambda t: (0, 0)),
            pl.BlockSpec((3 * H, H), lambda t: (0, 0)),
            pl.BlockSpec((2 * H, B), lambda t: (0, 0)),
            pl.BlockSpec((H, B), lambda t: (0, 0)),
            pl.BlockSpec((H, B), lambda t: (0, 0)),
        ],
        out_specs=pl.BlockSpec((H, B), lambda t: (0, 0)),
        out_shape=jax.ShapeDtypeStruct((H, B), jnp.float32),
        scratch_shapes=[pltpu.VMEM((H, B), jnp.float32)],
        interpret=interpret,
    )(emb, w_ih, w_hh, brz, bin_, bhn)


def kernel(x, table, W_ih, W_hh, b_ih, b_hh):
    emb = _sc_gather(table, x.astype(jnp.int32))
    brz = jnp.broadcast_to((b_ih[: 2 * H] + b_hh[: 2 * H])[:, None], (2 * H, B))
    bin_ = jnp.broadcast_to(b_ih[2 * H :][:, None], (H, B))
    bhn = jnp.broadcast_to(b_hh[2 * H :][:, None], (H, B))
    hn_t = _tc_gru(emb, W_ih, W_hh, brz, bin_, bhn)
    return hn_t.T[None]
